# baseline (speedup 1.0000x reference)
import functools

import jax
import jax.numpy as jnp
from jax import lax
from jax.experimental import pallas as pl
from jax.experimental.pallas import tpu as pltpu
from jax.experimental.pallas import tpu_sc as plsc

NC = 2
NS = 16
NW = NC * NS
LANES = 16
CHUNK = 128
BS = 8192


CCOLS = 1536


def _sc_project(tbl_t, wv):
  D, V = tbl_t.shape
  nfull = V // CCOLS
  tail = V - nfull * CCOLS
  npairs = (nfull + 2 * NW - 1) // (2 * NW)
  nblk = CCOLS // LANES

  mesh = plsc.VectorSubcoreMesh(core_axis_name="c", subcore_axis_name="s")

  @functools.partial(
      pl.kernel,
      out_type=jax.ShapeDtypeStruct((V,), jnp.float32),
      mesh=mesh,
      compiler_params=pltpu.CompilerParams(use_tc_tiling_on_sc=True),
      scratch_types=[
          pltpu.VMEM((D, CCOLS), jnp.float32),
          pltpu.VMEM((D, CCOLS), jnp.float32),
          pltpu.VMEM((D, 128), jnp.float32),
          pltpu.VMEM((D, tail if tail else 8), jnp.float32),
          pltpu.VMEM((CCOLS,), jnp.float32),
          pltpu.SemaphoreType.DMA,
          pltpu.SemaphoreType.DMA,
      ],
  )
  def k(tbl_hbm, w_hbm, p_hbm, xr0, xr1, w_v, xtail, p_v, sem0, sem1):
    wid = lax.axis_index("s") * NC + lax.axis_index("c")
    pltpu.sync_copy(w_hbm, w_v)

    def cid(j):
      return wid + NW * j

    def fire(j, xr, sem):
      @pl.when(cid(j) < nfull)
      def _():
        pltpu.async_copy(tbl_hbm.at[:, pl.ds(cid(j) * CCOLS, CCOLS)], xr,
                         sem)

    def wait(xr, sem):
      pltpu.make_async_copy(tbl_hbm.at[:, pl.ds(0, CCOLS)], xr, sem).wait()

    UNROLL = 4

    def axpy_blocks(xr, cols):
      def blk(b, carry):
        c0 = b * (UNROLL * LANES)
        accs = [jnp.zeros((LANES,), jnp.float32) for _ in range(UNROLL)]
        for d in range(D):
          wd = w_v[d, pl.ds(0, LANES)]
          accs = [
              a + xr[d, pl.ds(c0 + u * LANES, LANES)] * wd
              for u, a in enumerate(accs)
          ]
        for u, a in enumerate(accs):
          p_v[pl.ds(c0 + u * LANES, LANES)] = a
        return carry

      lax.fori_loop(0, cols // (UNROLL * LANES), blk, 0)

    def process(j, xr, sem):
      @pl.when(cid(j) < nfull)
      def _():
        wait(xr, sem)
        axpy_blocks(xr, CCOLS)
        pltpu.sync_copy(p_v, p_hbm.at[pl.ds(cid(j) * CCOLS, CCOLS)])

    fire(0, xr0, sem0)
    fire(1, xr1, sem1)

    def step(t, carry):
      j = 2 * t
      process(j, xr0, sem0)
      fire(j + 2, xr0, sem0)
      process(j + 1, xr1, sem1)
      fire(j + 3, xr1, sem1)
      return carry

    lax.fori_loop(0, npairs, step, 0)

    if tail:
      @pl.when(wid == 2)
      def _():
        pltpu.sync_copy(tbl_hbm.at[:, pl.ds(nfull * CCOLS, tail)], xtail)

        def blk(b, carry):
          c0 = b * LANES
          accv = jnp.zeros((LANES,), jnp.float32)
          for d in range(D):
            accv = accv + xtail[d, pl.ds(c0, LANES)] * w_v[d,
                                                           pl.ds(0, LANES)]
          p_v[pl.ds(c0, LANES)] = accv
          return carry

        lax.fori_loop(0, tail // LANES, blk, 0)
        pltpu.sync_copy(p_v.at[pl.ds(0, tail)],
                        p_hbm.at[pl.ds(nfull * CCOLS, tail)])

  return k(tbl_t, wv)


def _sc_pool(x128, p, b16, L):
  B = x128.shape[0]
  LC = (L + 7) // 8 * 8
  SKEW = LC + 1
  bpw = B // NW
  nchunk = bpw // CHUNK
  nblk = bpw // LANES

  mesh = plsc.VectorSubcoreMesh(core_axis_name="c", subcore_axis_name="s")

  @functools.partial(
      pl.kernel,
      out_type=jax.ShapeDtypeStruct((B,), jnp.float32),
      mesh=mesh,
      compiler_params=pltpu.CompilerParams(
          needs_layout_passes=False, use_tc_tiling_on_sc=False),
      scratch_types=[
          pltpu.VMEM((bpw, SKEW), jnp.int32),
          pltpu.VMEM((L, bpw), jnp.int32),
          pltpu.VMEM((bpw,), jnp.float32),
          pltpu.VMEM((bpw,), jnp.float32),
          pltpu.VMEM((bpw,), jnp.float32),
          pltpu.VMEM((bpw,), jnp.float32),
          pltpu.VMEM((bpw,), jnp.float32),
          pltpu.VMEM((LANES,), jnp.float32),
          pltpu.VMEM((bpw,), jnp.float32),
          pltpu.SemaphoreType.DMA,
          pltpu.SemaphoreType.DMA,
          pltpu.SemaphoreType.DMA,
          pltpu.SemaphoreType.DMA,
      ],
  )
  def k(x_hbm, p_hbm, b_hbm, out_hbm, xr_v, xt_v, g0, g1, g2, g3, acc, b_v,
        out_v, sem0, sem1, sem2, sem3):
    wid = lax.axis_index("s") * NC + lax.axis_index("c")
    base = wid * bpw
    pltpu.sync_copy(x_hbm.at[pl.ds(base, bpw), pl.ds(0, LC)],
                    xr_v.at[:, pl.ds(0, LC)])
    pltpu.sync_copy(b_hbm, b_v)

    lanes = lax.iota(jnp.int32, LANES)

    def tr(l, carry):
      col = jnp.broadcast_to(l, (LANES,))
      for blk in range(nblk):
        row = blk * LANES + lanes
        xt_v[l, pl.ds(blk * LANES, LANES)] = plsc.load_gather(
            xr_v, [row, col])
      return carry

    lax.fori_loop(0, L, tr, 0)

    zero = jnp.zeros((LANES,), jnp.float32)
    for blk in range(nblk):
      acc[pl.ds(blk * LANES, LANES)] = zero

    def fire(l, g, sem):
      for c in range(nchunk):
        idx = xt_v.at[l, pl.ds(c * CHUNK, CHUNK)]
        dst = g.at[pl.ds(c * CHUNK, CHUNK)]
        pltpu.async_copy(p_hbm.at[idx], dst, sem)

    def drain(g, sem):
      pltpu.make_async_copy(p_hbm.at[pl.ds(0, bpw)], g, sem).wait()

    def accumulate(g):
      for blk in range(nblk):
        o = blk * LANES
        acc[pl.ds(o, LANES)] = acc[pl.ds(o, LANES)] + g[pl.ds(o, LANES)]

    bufs = ((g0, sem0), (g1, sem1), (g2, sem2), (g3, sem3))
    for l0 in range(min(4, L)):
      fire(l0, *bufs[l0])

    def step(t, carry):
      for ph in range(4):
        l = 4 * t + ph
        g, sem = bufs[ph]

        @pl.when(l < L)
        def _():
          drain(g, sem)
          accumulate(g)

        @pl.when(l + 4 < L)
        def _():
          fire(l + 4, g, sem)

      return carry

    lax.fori_loop(0, (L + 3) // 4, step, 0)

    inv_l = jnp.float32(1.0 / L)
    bias_vec = b_v[pl.ds(0, LANES)]
    for blk in range(nblk):
      o = blk * LANES
      out_v[pl.ds(o, LANES)] = acc[pl.ds(o, LANES)] * inv_l + bias_vec

    pltpu.sync_copy(out_v, out_hbm.at[pl.ds(base, bpw)])

  return k(x128, p, b16)


@jax.jit
def _prep_and_run(x, table, W, b):
  B, L = x.shape
  x128 = jnp.pad(x.astype(jnp.int32), ((0, 0), (0, 128 - L)))
  w = W.reshape(-1).astype(jnp.float32)
  wv = jnp.broadcast_to(w[:, None], (w.shape[0], 128))
  b16 = jnp.broadcast_to(b.reshape(-1)[:1], (LANES,)).astype(jnp.float32)
  tbl_t = jnp.transpose(table)
  p = _sc_project(tbl_t, wv)
  return _sc_pool(x128, p, b16, L)


def kernel(x, table, W, b):
  return _prep_and_run(x, table, W, b)


# speedup vs baseline: 1.0264x; 1.0264x over previous
import functools

import jax
import jax.numpy as jnp
from jax import lax
from jax.experimental import pallas as pl
from jax.experimental.pallas import tpu as pltpu
from jax.experimental.pallas import tpu_sc as plsc

NC = 2
NS = 16
NW = NC * NS
LANES = 16
CHUNK = 128
BS = 8192


CCOLS = 1536


def _sc_project(tbl_t, wv):
  D, V = tbl_t.shape
  nfull = V // CCOLS
  tail = V - nfull * CCOLS
  npairs = (nfull + 2 * NW - 1) // (2 * NW)
  nblk = CCOLS // LANES

  mesh = plsc.VectorSubcoreMesh(core_axis_name="c", subcore_axis_name="s")

  @functools.partial(
      pl.kernel,
      out_type=jax.ShapeDtypeStruct((V,), jnp.float32),
      mesh=mesh,
      compiler_params=pltpu.CompilerParams(use_tc_tiling_on_sc=True),
      scratch_types=[
          pltpu.VMEM((D, CCOLS), jnp.float32),
          pltpu.VMEM((D, CCOLS), jnp.float32),
          pltpu.VMEM((D, 128), jnp.float32),
          pltpu.VMEM((D, tail if tail else 8), jnp.float32),
          pltpu.VMEM((CCOLS,), jnp.float32),
          pltpu.SemaphoreType.DMA,
          pltpu.SemaphoreType.DMA,
      ],
  )
  def k(tbl_hbm, w_hbm, p_hbm, xr0, xr1, w_v, xtail, p_v, sem0, sem1):
    wid = lax.axis_index("s") * NC + lax.axis_index("c")
    pltpu.sync_copy(w_hbm, w_v)

    def cid(j):
      return wid + NW * j

    def fire(j, xr, sem):
      @pl.when(cid(j) < nfull)
      def _():
        pltpu.async_copy(tbl_hbm.at[:, pl.ds(cid(j) * CCOLS, CCOLS)], xr,
                         sem)

    def wait(xr, sem):
      pltpu.make_async_copy(tbl_hbm.at[:, pl.ds(0, CCOLS)], xr, sem).wait()

    UNROLL = 2

    def axpy_blocks(xr, cols):
      def blk(b, carry):
        c0 = b * (UNROLL * LANES)
        accs = [jnp.zeros((LANES,), jnp.float32) for _ in range(UNROLL)]
        for d in range(D):
          wd = w_v[d, pl.ds(0, LANES)]
          accs = [
              a + xr[d, pl.ds(c0 + u * LANES, LANES)] * wd
              for u, a in enumerate(accs)
          ]
        for u, a in enumerate(accs):
          p_v[pl.ds(c0 + u * LANES, LANES)] = a
        return carry

      lax.fori_loop(0, cols // (UNROLL * LANES), blk, 0)

    def process(j, xr, sem):
      @pl.when(cid(j) < nfull)
      def _():
        wait(xr, sem)
        axpy_blocks(xr, CCOLS)
        pltpu.sync_copy(p_v, p_hbm.at[pl.ds(cid(j) * CCOLS, CCOLS)])

    fire(0, xr0, sem0)
    fire(1, xr1, sem1)

    def step(t, carry):
      j = 2 * t
      process(j, xr0, sem0)
      fire(j + 2, xr0, sem0)
      process(j + 1, xr1, sem1)
      fire(j + 3, xr1, sem1)
      return carry

    lax.fori_loop(0, npairs, step, 0)

    if tail:
      @pl.when(wid == 2)
      def _():
        pltpu.sync_copy(tbl_hbm.at[:, pl.ds(nfull * CCOLS, tail)], xtail)

        def blk(b, carry):
          c0 = b * LANES
          accv = jnp.zeros((LANES,), jnp.float32)
          for d in range(D):
            accv = accv + xtail[d, pl.ds(c0, LANES)] * w_v[d,
                                                           pl.ds(0, LANES)]
          p_v[pl.ds(c0, LANES)] = accv
          return carry

        lax.fori_loop(0, tail // LANES, blk, 0)
        pltpu.sync_copy(p_v.at[pl.ds(0, tail)],
                        p_hbm.at[pl.ds(nfull * CCOLS, tail)])

  return k(tbl_t, wv)


def _sc_pool(x128, p, b16, L):
  B = x128.shape[0]
  LC = (L + 7) // 8 * 8
  SKEW = LC + 1
  bpw = B // NW
  nchunk = bpw // CHUNK
  nblk = bpw // LANES

  mesh = plsc.VectorSubcoreMesh(core_axis_name="c", subcore_axis_name="s")

  @functools.partial(
      pl.kernel,
      out_type=jax.ShapeDtypeStruct((B,), jnp.float32),
      mesh=mesh,
      compiler_params=pltpu.CompilerParams(
          needs_layout_passes=False, use_tc_tiling_on_sc=False),
      scratch_types=[
          pltpu.VMEM((bpw, SKEW), jnp.int32),
          pltpu.VMEM((L, bpw), jnp.int32),
          pltpu.VMEM((bpw,), jnp.float32),
          pltpu.VMEM((bpw,), jnp.float32),
          pltpu.VMEM((bpw,), jnp.float32),
          pltpu.VMEM((bpw,), jnp.float32),
          pltpu.VMEM((bpw,), jnp.float32),
          pltpu.VMEM((LANES,), jnp.float32),
          pltpu.VMEM((bpw,), jnp.float32),
          pltpu.SemaphoreType.DMA,
          pltpu.SemaphoreType.DMA,
          pltpu.SemaphoreType.DMA,
          pltpu.SemaphoreType.DMA,
      ],
  )
  def k(x_hbm, p_hbm, b_hbm, out_hbm, xr_v, xt_v, g0, g1, g2, g3, acc, b_v,
        out_v, sem0, sem1, sem2, sem3):
    wid = lax.axis_index("s") * NC + lax.axis_index("c")
    base = wid * bpw
    pltpu.sync_copy(x_hbm.at[pl.ds(base, bpw), pl.ds(0, LC)],
                    xr_v.at[:, pl.ds(0, LC)])
    pltpu.sync_copy(b_hbm, b_v)

    lanes = lax.iota(jnp.int32, LANES)

    def tr(l, carry):
      col = jnp.broadcast_to(l, (LANES,))
      for blk in range(nblk):
        row = blk * LANES + lanes
        xt_v[l, pl.ds(blk * LANES, LANES)] = plsc.load_gather(
            xr_v, [row, col])
      return carry

    lax.fori_loop(0, L, tr, 0)

    zero = jnp.zeros((LANES,), jnp.float32)
    for blk in range(nblk):
      acc[pl.ds(blk * LANES, LANES)] = zero

    def fire(l, g, sem):
      for c in range(nchunk):
        idx = xt_v.at[l, pl.ds(c * CHUNK, CHUNK)]
        dst = g.at[pl.ds(c * CHUNK, CHUNK)]
        pltpu.async_copy(p_hbm.at[idx], dst, sem)

    def drain(g, sem):
      pltpu.make_async_copy(p_hbm.at[pl.ds(0, bpw)], g, sem).wait()

    def accumulate(g):
      for blk in range(nblk):
        o = blk * LANES
        acc[pl.ds(o, LANES)] = acc[pl.ds(o, LANES)] + g[pl.ds(o, LANES)]

    bufs = ((g0, sem0), (g1, sem1), (g2, sem2), (g3, sem3))
    for l0 in range(min(4, L)):
      fire(l0, *bufs[l0])

    def step(t, carry):
      for ph in range(4):
        l = 4 * t + ph
        g, sem = bufs[ph]

        @pl.when(l < L)
        def _():
          drain(g, sem)
          accumulate(g)

        @pl.when(l + 4 < L)
        def _():
          fire(l + 4, g, sem)

      return carry

    lax.fori_loop(0, (L + 3) // 4, step, 0)

    inv_l = jnp.float32(1.0 / L)
    bias_vec = b_v[pl.ds(0, LANES)]
    for blk in range(nblk):
      o = blk * LANES
      out_v[pl.ds(o, LANES)] = acc[pl.ds(o, LANES)] * inv_l + bias_vec

    pltpu.sync_copy(out_v, out_hbm.at[pl.ds(base, bpw)])

  return k(x128, p, b16)


@jax.jit
def _prep_and_run(x, table, W, b):
  B, L = x.shape
  x128 = jnp.pad(x.astype(jnp.int32), ((0, 0), (0, 128 - L)))
  w = W.reshape(-1).astype(jnp.float32)
  wv = jnp.broadcast_to(w[:, None], (w.shape[0], 128))
  b16 = jnp.broadcast_to(b.reshape(-1)[:1], (LANES,)).astype(jnp.float32)
  tbl_t = jnp.transpose(table)
  p = _sc_project(tbl_t, wv)
  return _sc_pool(x128, p, b16, L)


def kernel(x, table, W, b):
  return _prep_and_run(x, table, W, b)
